# baseline (speedup 1.0000x reference)
import functools

import jax
import jax.numpy as jnp
from jax import lax
from jax.experimental import pallas as pl
from jax.experimental.pallas import tpu as pltpu
from jax.experimental.pallas import tpu_sc as plsc

N, E, B = 10000, 320000, 16
EBLK = 512
NBE = E // EBLK
NP = 10240
NBN = NP // 128
NW = 32
EPW = E // NW
GCH = 80
NCH_G = EPW // GCH
DCH = 4000
NCH_D = E // DCH
NODES_PW = NP // NW

NEG_INF = float("-inf")

_sc_mesh = lambda: plsc.VectorSubcoreMesh(
    core_axis_name="c", subcore_axis_name="s")


def _wid():
    return lax.axis_index("s") * 2 + lax.axis_index("c")


def _sc_rel_body(pos_hbm, src_hbm, dst_hbm, rel_hbm, postbl, sidx, didx, rbuf):
    base = _wid() * EPW
    pltpu.sync_copy(pos_hbm, postbl)

    def zgrp(gi, _):
        rbuf[pl.ds(gi * 16, 16)] = jnp.zeros((16,), jnp.float32)
        return 0

    lax.fori_loop(0, (GCH * 8) // 16, zgrp, 0)

    def chunk(ci, _):
        off = base + ci * GCH
        pltpu.sync_copy(src_hbm.at[pl.ds(off, GCH)], sidx)
        pltpu.sync_copy(dst_hbm.at[pl.ds(off, GCH)], didx)

        def grp(gi, _):
            ls = sidx[pl.ds(gi * 16, 16)] * 8
            ld = didx[pl.ds(gi * 16, 16)] * 8
            for k in range(3):
                sv = plsc.load_gather(postbl, [ls + k])
                dv = plsc.load_gather(postbl, [ld + k])
                fidx = jnp.arange(16, dtype=jnp.int32) * 8 + (gi * 128 + k)
                plsc.store_scatter(rbuf, [fidx], sv - dv)
            return 0

        lax.fori_loop(0, GCH // 16, grp, 0)
        pltpu.sync_copy(rbuf, rel_hbm.at[pl.ds(off * 8, GCH * 8)])
        return 0

    lax.fori_loop(0, NCH_G, chunk, 0)


def _sc_rel(posp, src, dst):
    f = functools.partial(
        pl.kernel,
        out_type=jax.ShapeDtypeStruct((E * 8,), jnp.float32),
        mesh=_sc_mesh(),
        compiler_params=pltpu.CompilerParams(needs_layout_passes=False),
        scratch_types=[
            pltpu.VMEM((N * 8,), jnp.float32),
            pltpu.VMEM((GCH,), jnp.int32),
            pltpu.VMEM((GCH,), jnp.int32),
            pltpu.VMEM((GCH * 8,), jnp.float32),
        ],
    )(_sc_rel_body)
    return f(posp, src, dst)


def _sc_segment_max_body(h_hbm, dst_hbm, x_hbm, iso_hbm,
                         dbuf, cidx, cdst, hrows, acc, ibuf, sem):
    wid = _wid()
    lo = wid * NODES_PW
    hi = lo + NODES_PW

    def initrow(r, _):
        for k in range(8):
            acc[r, pl.ds(k * 16, 16)] = jnp.full((16,), NEG_INF, jnp.float32)
        return 0

    lax.fori_loop(0, NODES_PW, initrow, 0)

    def initcidx(gi, _):
        cidx[pl.ds(gi * 16, 16)] = jnp.zeros((16,), jnp.int32)
        cdst[pl.ds(gi * 16, 16)] = jnp.zeros((16,), jnp.int32)
        return 0

    lax.fori_loop(0, 4096 // 16, initcidx, 0)

    def chunk(ci, _):
        eoff = ci * DCH
        pltpu.sync_copy(dst_hbm.at[pl.ds(eoff, DCH)], dbuf)

        def grp(gi, cnt):
            d = dbuf[pl.ds(gi * 16, 16)]
            m = (d >= lo) & (d < hi)
            eid = jnp.arange(16, dtype=jnp.int32) + (eoff + gi * 16)
            plsc.store_compressed(cidx.at[pl.ds(cnt, 16)], eid, mask=m)
            plsc.store_compressed(cdst.at[pl.ds(cnt, 16)], d - lo, mask=m)
            return cnt + jnp.sum(m.astype(jnp.int32))

        cnt = lax.fori_loop(0, DCH // 16, grp, 0)

        def piece(pi, _):
            pltpu.async_copy(
                h_hbm.at[cidx.at[pl.ds(pi * 128, 128)]], hrows, sem).wait()
            mj = jnp.minimum(cnt - pi * 128, 128)

            def rmw(j, _):
                nrow = cdst[pl.ds(pi * 128 + j, 16)][0]
                for k in range(8):
                    sl = pl.ds(k * 16, 16)
                    acc[nrow, sl] = jnp.maximum(acc[nrow, sl], hrows[j, sl])
                return 0

            lax.fori_loop(0, mj, rmw, 0)
            return 0

        lax.fori_loop(0, (cnt + 127) // 128, piece, 0)
        return 0

    lax.fori_loop(0, NCH_D, chunk, 0)

    def fingrp(gi, _):
        rvec = jnp.arange(16, dtype=jnp.int32) + gi * 16
        vals = plsc.load_gather(acc, [rvec, jnp.zeros((16,), jnp.int32)])
        flags = jnp.where(vals == NEG_INF, 1, 0).astype(jnp.int32)
        ibuf[pl.ds(gi * 16, 16)] = flags
        return 0

    lax.fori_loop(0, NODES_PW // 16, fingrp, 0)

    def finrow(r, _):
        for k in range(8):
            sl = pl.ds(k * 16, 16)
            w = acc[r, sl]
            acc[r, sl] = jnp.where(w == NEG_INF, 0.0, w)
        return 0

    lax.fori_loop(0, NODES_PW, finrow, 0)
    pltpu.sync_copy(acc, x_hbm.at[pl.ds(lo, NODES_PW)])
    pltpu.sync_copy(ibuf, iso_hbm.at[pl.ds(lo, NODES_PW)])


def _sc_segment_max(h, dst):
    f = functools.partial(
        pl.kernel,
        out_type=[jax.ShapeDtypeStruct((NP, 128), jnp.float32),
                  jax.ShapeDtypeStruct((NP,), jnp.int32)],
        mesh=_sc_mesh(),
        compiler_params=pltpu.CompilerParams(needs_layout_passes=False),
        scratch_types=[
            pltpu.VMEM((DCH,), jnp.int32),
            pltpu.VMEM((4096,), jnp.int32),
            pltpu.VMEM((4096,), jnp.int32),
            pltpu.VMEM((128, 128), jnp.float32),
            pltpu.VMEM((NODES_PW, 128), jnp.float32),
            pltpu.VMEM((NODES_PW,), jnp.int32),
            pltpu.SemaphoreType.DMA,
        ],
    )(_sc_segment_max_body)
    return f(h, dst)


def _sc_gather_u2_body(u2_hbm, src_hbm, us_hbm, sidx, ubuf, sem):
    base = _wid() * EPW

    def chunk(ci, _):
        off = base + ci * GCH
        pltpu.sync_copy(src_hbm.at[pl.ds(off, GCH)], sidx)
        pltpu.async_copy(u2_hbm.at[sidx], ubuf, sem).wait()
        pltpu.sync_copy(ubuf, us_hbm.at[pl.ds(off, GCH)])
        return 0

    lax.fori_loop(0, NCH_G, chunk, 0)


def _sc_gather_u2(u2, src):
    f = functools.partial(
        pl.kernel,
        out_type=jax.ShapeDtypeStruct((E, 256), jnp.float32),
        mesh=_sc_mesh(),
        compiler_params=pltpu.CompilerParams(needs_layout_passes=False),
        scratch_types=[
            pltpu.VMEM((GCH,), jnp.int32),
            pltpu.VMEM((GCH, 256), jnp.float32),
            pltpu.SemaphoreType.DMA,
        ],
    )(_sc_gather_u2_body)
    return f(u2, src)


def _mlp1_body(rel_ref, w1a_ref, b1a_ref, w1b_ref, b1b_ref, out_ref):
    relb = rel_ref[...]
    t = jnp.dot(relb, w1a_ref[...], preferred_element_type=jnp.float32)
    t = jnp.maximum(t + b1a_ref[...], 0.0)
    h = jnp.dot(t, w1b_ref[...], preferred_element_type=jnp.float32)
    out_ref[...] = h + b1b_ref[...]


def _mlp1(rel, w1a8, b1a, w1b, b1b):
    return pl.pallas_call(
        _mlp1_body,
        grid=(NBE,),
        in_specs=[
            pl.BlockSpec((EBLK, 8), lambda i: (i, 0)),
            pl.BlockSpec((8, 64), lambda i: (0, 0)),
            pl.BlockSpec((1, 64), lambda i: (0, 0)),
            pl.BlockSpec((64, 128), lambda i: (0, 0)),
            pl.BlockSpec((1, 128), lambda i: (0, 0)),
        ],
        out_specs=pl.BlockSpec((EBLK, 128), lambda i: (i, 0)),
        out_shape=jax.ShapeDtypeStruct((E, 128), jnp.float32),
    )(rel, w1a8, b1a, w1b, b1b)


def _tables_body(x_ref, w2ax_ref, b2a_ref, u_ref):
    ub = jnp.dot(x_ref[...], w2ax_ref[...], preferred_element_type=jnp.float32)
    u_ref[...] = ub + b2a_ref[...]


def _tables(xp, w2ax, b2a):
    return pl.pallas_call(
        _tables_body,
        grid=(NBN,),
        in_specs=[
            pl.BlockSpec((128, 128), lambda i: (i, 0)),
            pl.BlockSpec((128, 256), lambda i: (0, 0)),
            pl.BlockSpec((1, 256), lambda i: (0, 0)),
        ],
        out_specs=pl.BlockSpec((128, 256), lambda i: (i, 0)),
        out_shape=jax.ShapeDtypeStruct((NP, 256), jnp.float32),
    )(xp, w2ax, b2a)


def _mlp2_body(us_ref, rel_ref, dst_ref, batch_ref, iso_ref,
               w2ar_ref, w2b_ref, b2b_ref, g_ref):
    i = pl.program_id(0)

    @pl.when(i == 0)
    def _init():
        g_ref[...] = jnp.full((B, 512), NEG_INF, jnp.float32)

    relb = rel_ref[...]
    a = us_ref[...] + jnp.dot(relb, w2ar_ref[...],
                              preferred_element_type=jnp.float32)
    v = jnp.maximum(a, 0.0)
    h2 = jnp.dot(v, w2b_ref[...], preferred_element_type=jnp.float32)
    h2 = h2 + b2b_ref[...]
    dst = dst_ref[0]
    batch = batch_ref[...]

    lbl = jnp.zeros_like(dst)
    for b in range(1, B):
        start_b = jnp.sum((batch < b).astype(jnp.int32))
        lbl = lbl + (dst >= start_b).astype(jnp.int32)

    acc = g_ref[...]
    rows = []
    for b in range(B):
        mb = jnp.max(jnp.where(lbl == b, h2, NEG_INF), axis=0, keepdims=True)
        rows.append(mb)
    g_ref[...] = jnp.maximum(acc, jnp.concatenate(rows, axis=0))

    @pl.when(i == NBE - 1)
    def _fin():
        iso = iso_ref[...]
        g = g_ref[...]
        floors = []
        for b in range(B):
            has_iso = jnp.sum(iso * (batch == b).astype(jnp.int32)) > 0
            floors.append(jnp.where(has_iso, 0.0, NEG_INF).reshape(1, 1))
        g = jnp.maximum(g, jnp.concatenate(floors, axis=0))
        g_ref[...] = jnp.where(jnp.isfinite(g), g, 0.0)


def _mlp2_pool(us, rel, dst3, batchp, isop, w2ar8, w2b, b2b):
    return pl.pallas_call(
        _mlp2_body,
        grid=(NBE,),
        in_specs=[
            pl.BlockSpec((EBLK, 256), lambda i: (i, 0)),
            pl.BlockSpec((EBLK, 8), lambda i: (i, 0)),
            pl.BlockSpec((1, EBLK, 1), lambda i: (i, 0, 0)),
            pl.BlockSpec((80, 128), lambda i: (0, 0)),
            pl.BlockSpec((80, 128), lambda i: (0, 0)),
            pl.BlockSpec((8, 256), lambda i: (0, 0)),
            pl.BlockSpec((256, 512), lambda i: (0, 0)),
            pl.BlockSpec((1, 512), lambda i: (0, 0)),
        ],
        out_specs=pl.BlockSpec((B, 512), lambda i: (0, 0)),
        out_shape=jax.ShapeDtypeStruct((B, 512), jnp.float32),
    )(us, rel, dst3, batchp, isop, w2ar8, w2b, b2b)


def kernel(pos, edge_index, batch, W1a, b1a, W1b, b1b, W2a, b2a, W2b, b2b):
    src = edge_index[0]
    dst = edge_index[1]

    posp = jnp.zeros((N, 8), jnp.float32).at[:, :3].set(pos).reshape(N * 8)
    w1a8 = jnp.zeros((8, 64), jnp.float32).at[:3].set(W1a)
    w2ar8 = jnp.zeros((8, 256), jnp.float32).at[:3].set(W2a[128:131])

    rel = _sc_rel(posp, src, dst).reshape(E, 8)

    h = _mlp1(rel, w1a8, b1a.reshape(1, 64), W1b, b1b.reshape(1, 128))

    xp, iso = _sc_segment_max(h, dst)

    u2 = _tables(xp, W2a[:128], b2a.reshape(1, 256))

    us = _sc_gather_u2(u2, src)

    batchp = jnp.full((NP,), B, jnp.int32).at[:N].set(batch).reshape(80, 128)
    isop = iso.reshape(80, 128)
    dst3 = dst.reshape(NBE, EBLK, 1)

    return _mlp2_pool(us, rel, dst3, batchp, isop, w2ar8, W2b,
                      b2b.reshape(1, 512))


# speedup vs baseline: 1.5377x; 1.5377x over previous
import functools

import jax
import jax.numpy as jnp
from jax import lax
from jax.experimental import pallas as pl
from jax.experimental.pallas import tpu as pltpu
from jax.experimental.pallas import tpu_sc as plsc

N, E, B = 10000, 320000, 16
EBLK = 512
NBE = E // EBLK
NP = 10240
NBN = NP // 128
NW = 32
EPW = E // NW
GCH = 80
NCH_G = EPW // GCH
DCH = 8000
NCH_D = E // DCH
NODES_PW = NP // NW

NEG_INF = float("-inf")

_sc_mesh = lambda: plsc.VectorSubcoreMesh(
    core_axis_name="c", subcore_axis_name="s")


def _wid():
    return lax.axis_index("s") * 2 + lax.axis_index("c")


def _sc_rel_body(pos_hbm, src_hbm, dst_hbm, rel_hbm, postbl, sidx, didx, rbuf):
    base = _wid() * EPW
    pltpu.sync_copy(pos_hbm, postbl)

    def zgrp(gi, _):
        rbuf[pl.ds(gi * 16, 16)] = jnp.zeros((16,), jnp.float32)
        return 0

    lax.fori_loop(0, (GCH * 8) // 16, zgrp, 0)

    def chunk(ci, _):
        off = base + ci * GCH
        pltpu.sync_copy(src_hbm.at[pl.ds(off, GCH)], sidx)
        pltpu.sync_copy(dst_hbm.at[pl.ds(off, GCH)], didx)

        def grp(gi, _):
            ls = sidx[pl.ds(gi * 16, 16)] * 8
            ld = didx[pl.ds(gi * 16, 16)] * 8
            for k in range(3):
                sv = plsc.load_gather(postbl, [ls + k])
                dv = plsc.load_gather(postbl, [ld + k])
                fidx = jnp.arange(16, dtype=jnp.int32) * 8 + (gi * 128 + k)
                plsc.store_scatter(rbuf, [fidx], sv - dv)
            return 0

        lax.fori_loop(0, GCH // 16, grp, 0)
        pltpu.sync_copy(rbuf, rel_hbm.at[pl.ds(off * 8, GCH * 8)])
        return 0

    lax.fori_loop(0, NCH_G, chunk, 0)


def _sc_rel(posp, src, dst):
    f = functools.partial(
        pl.kernel,
        out_type=jax.ShapeDtypeStruct((E * 8,), jnp.float32),
        mesh=_sc_mesh(),
        compiler_params=pltpu.CompilerParams(needs_layout_passes=False),
        scratch_types=[
            pltpu.VMEM((N * 8,), jnp.float32),
            pltpu.VMEM((GCH,), jnp.int32),
            pltpu.VMEM((GCH,), jnp.int32),
            pltpu.VMEM((GCH * 8,), jnp.float32),
        ],
    )(_sc_rel_body)
    return f(posp, src, dst)


KMASK = (1 << 19) - 1


def _sc_segment_max_body(h_hbm, dst_hbm, x_hbm, iso_hbm,
                         dbuf0, dbuf1, cidx, gidx, hrows, acc, ibuf,
                         dsem0, dsem1, gsem):
    wid = _wid()
    lo = wid * NODES_PW
    hi = lo + NODES_PW

    def initrow(r, _):
        for k in range(8):
            acc[r, pl.ds(k * 16, 16)] = jnp.full((16,), NEG_INF, jnp.float32)
        return 0

    lax.fori_loop(0, NODES_PW, initrow, 0)

    def initcidx(gi, _):
        cidx[pl.ds(gi * 16, 16)] = jnp.zeros((16,), jnp.int32)
        return 0

    lax.fori_loop(0, (DCH + 128) // 16, initcidx, 0)

    dbufs = (dbuf0, dbuf1)
    dsems = (dsem0, dsem1)
    pltpu.async_copy(dst_hbm.at[pl.ds(0, DCH)], dbuf0, dsem0)
    pltpu.async_copy(dst_hbm.at[pl.ds(DCH, DCH)], dbuf1, dsem1)

    def pair(cj, _):
        for b in range(2):
            ci = cj * 2 + b
            eoff = ci * DCH
            dbufb = dbufs[b]
            pltpu.make_async_copy(
                dst_hbm.at[pl.ds(eoff, DCH)], dbufb, dsems[b]).wait()

            def grp(gi, cnt):
                d = dbufb[pl.ds(gi * 16, 16)]
                m = (d >= lo) & (d < hi)
                eid = jnp.arange(16, dtype=jnp.int32) + (eoff + gi * 16)
                key = ((d - lo) << 19) | eid
                plsc.store_compressed(cidx.at[pl.ds(cnt, 16)], key, mask=m)
                return cnt + plsc.all_reduce_population_count(m)[0]

            cnt = lax.fori_loop(0, DCH // 16, grp, 0)

            @pl.when(ci + 2 < NCH_D)
            def _prefetch():
                pltpu.async_copy(
                    dst_hbm.at[pl.ds(eoff + 2 * DCH, DCH)], dbufb, dsems[b])

            npieces = (cnt + 127) // 128

            def conv(gi, _):
                kv = cidx[pl.ds(gi * 16, 16)]
                gidx[pl.ds(gi * 16, 16)] = kv & KMASK
                return 0

            lax.fori_loop(0, npieces * 8, conv, 0)

            def piece(pi, _):
                pltpu.async_copy(
                    h_hbm.at[gidx.at[pl.ds(pi * 128, 128)]], hrows,
                    gsem).wait()
                mj = jnp.minimum(cnt - pi * 128, 128)

                def rmw(j, _):
                    key = cidx[pl.ds(pi * 128 + j, 16)][0]
                    nrow = key >> 19
                    for k in range(8):
                        sl = pl.ds(k * 16, 16)
                        acc[nrow, sl] = jnp.maximum(acc[nrow, sl],
                                                    hrows[j, sl])
                    return 0

                lax.fori_loop(0, mj, rmw, 0)
                return 0

            lax.fori_loop(0, npieces, piece, 0)
        return 0

    lax.fori_loop(0, NCH_D // 2, pair, 0)

    def fingrp(gi, _):
        rvec = jnp.arange(16, dtype=jnp.int32) + gi * 16
        vals = plsc.load_gather(acc, [rvec, jnp.zeros((16,), jnp.int32)])
        flags = jnp.where(vals == NEG_INF, 1, 0).astype(jnp.int32)
        ibuf[pl.ds(gi * 16, 16)] = flags
        return 0

    lax.fori_loop(0, NODES_PW // 16, fingrp, 0)

    def finrow(r, _):
        for k in range(8):
            sl = pl.ds(k * 16, 16)
            w = acc[r, sl]
            acc[r, sl] = jnp.where(w == NEG_INF, 0.0, w)
        return 0

    lax.fori_loop(0, NODES_PW, finrow, 0)
    pltpu.sync_copy(acc, x_hbm.at[pl.ds(lo, NODES_PW)])
    pltpu.sync_copy(ibuf, iso_hbm.at[pl.ds(lo, NODES_PW)])


def _sc_segment_max(h, dst):
    f = functools.partial(
        pl.kernel,
        out_type=[jax.ShapeDtypeStruct((NP, 128), jnp.float32),
                  jax.ShapeDtypeStruct((NP,), jnp.int32)],
        mesh=_sc_mesh(),
        compiler_params=pltpu.CompilerParams(needs_layout_passes=False),
        scratch_types=[
            pltpu.VMEM((DCH,), jnp.int32),
            pltpu.VMEM((DCH,), jnp.int32),
            pltpu.VMEM((DCH + 128,), jnp.int32),
            pltpu.VMEM((DCH + 128,), jnp.int32),
            pltpu.VMEM((128, 128), jnp.float32),
            pltpu.VMEM((NODES_PW, 128), jnp.float32),
            pltpu.VMEM((NODES_PW,), jnp.int32),
            pltpu.SemaphoreType.DMA,
            pltpu.SemaphoreType.DMA,
            pltpu.SemaphoreType.DMA,
        ],
    )(_sc_segment_max_body)
    return f(h, dst)


def _sc_gather_u2_body(u2_hbm, src_hbm, us_hbm, sidx, ubuf, sem):
    base = _wid() * EPW

    def chunk(ci, _):
        off = base + ci * GCH
        pltpu.sync_copy(src_hbm.at[pl.ds(off, GCH)], sidx)
        pltpu.async_copy(u2_hbm.at[sidx], ubuf, sem).wait()
        pltpu.sync_copy(ubuf, us_hbm.at[pl.ds(off, GCH)])
        return 0

    lax.fori_loop(0, NCH_G, chunk, 0)


def _sc_gather_u2(u2, src):
    f = functools.partial(
        pl.kernel,
        out_type=jax.ShapeDtypeStruct((E, 256), jnp.float32),
        mesh=_sc_mesh(),
        compiler_params=pltpu.CompilerParams(needs_layout_passes=False),
        scratch_types=[
            pltpu.VMEM((GCH,), jnp.int32),
            pltpu.VMEM((GCH, 256), jnp.float32),
            pltpu.SemaphoreType.DMA,
        ],
    )(_sc_gather_u2_body)
    return f(u2, src)


def _mlp1_body(rel_ref, w1a_ref, b1a_ref, w1b_ref, b1b_ref, out_ref):
    relb = rel_ref[...]
    t = jnp.dot(relb, w1a_ref[...], preferred_element_type=jnp.float32)
    t = jnp.maximum(t + b1a_ref[...], 0.0)
    h = jnp.dot(t, w1b_ref[...], preferred_element_type=jnp.float32)
    out_ref[...] = h + b1b_ref[...]


def _mlp1(rel, w1a8, b1a, w1b, b1b):
    return pl.pallas_call(
        _mlp1_body,
        grid=(NBE,),
        in_specs=[
            pl.BlockSpec((EBLK, 8), lambda i: (i, 0)),
            pl.BlockSpec((8, 64), lambda i: (0, 0)),
            pl.BlockSpec((1, 64), lambda i: (0, 0)),
            pl.BlockSpec((64, 128), lambda i: (0, 0)),
            pl.BlockSpec((1, 128), lambda i: (0, 0)),
        ],
        out_specs=pl.BlockSpec((EBLK, 128), lambda i: (i, 0)),
        out_shape=jax.ShapeDtypeStruct((E, 128), jnp.float32),
    )(rel, w1a8, b1a, w1b, b1b)


def _tables_body(x_ref, w2ax_ref, b2a_ref, u_ref):
    ub = jnp.dot(x_ref[...], w2ax_ref[...], preferred_element_type=jnp.float32)
    u_ref[...] = ub + b2a_ref[...]


def _tables(xp, w2ax, b2a):
    return pl.pallas_call(
        _tables_body,
        grid=(NBN,),
        in_specs=[
            pl.BlockSpec((128, 128), lambda i: (i, 0)),
            pl.BlockSpec((128, 256), lambda i: (0, 0)),
            pl.BlockSpec((1, 256), lambda i: (0, 0)),
        ],
        out_specs=pl.BlockSpec((128, 256), lambda i: (i, 0)),
        out_shape=jax.ShapeDtypeStruct((NP, 256), jnp.float32),
    )(xp, w2ax, b2a)


def _mlp2_body(us_ref, rel_ref, dst_ref, batch_ref, iso_ref,
               w2ar_ref, w2b_ref, b2b_ref, g_ref):
    i = pl.program_id(0)

    @pl.when(i == 0)
    def _init():
        g_ref[...] = jnp.full((B, 512), NEG_INF, jnp.float32)

    relb = rel_ref[...]
    a = us_ref[...] + jnp.dot(relb, w2ar_ref[...],
                              preferred_element_type=jnp.float32)
    v = jnp.maximum(a, 0.0)
    h2 = jnp.dot(v, w2b_ref[...], preferred_element_type=jnp.float32)
    h2 = h2 + b2b_ref[...]
    dst = dst_ref[0]
    batch = batch_ref[...]

    lbl = jnp.zeros_like(dst)
    for b in range(1, B):
        start_b = jnp.sum((batch < b).astype(jnp.int32))
        lbl = lbl + (dst >= start_b).astype(jnp.int32)

    acc = g_ref[...]
    rows = []
    for b in range(B):
        mb = jnp.max(jnp.where(lbl == b, h2, NEG_INF), axis=0, keepdims=True)
        rows.append(mb)
    g_ref[...] = jnp.maximum(acc, jnp.concatenate(rows, axis=0))

    @pl.when(i == NBE - 1)
    def _fin():
        iso = iso_ref[...]
        g = g_ref[...]
        floors = []
        for b in range(B):
            has_iso = jnp.sum(iso * (batch == b).astype(jnp.int32)) > 0
            floors.append(jnp.where(has_iso, 0.0, NEG_INF).reshape(1, 1))
        g = jnp.maximum(g, jnp.concatenate(floors, axis=0))
        g_ref[...] = jnp.where(jnp.isfinite(g), g, 0.0)


def _mlp2_pool(us, rel, dst3, batchp, isop, w2ar8, w2b, b2b):
    return pl.pallas_call(
        _mlp2_body,
        grid=(NBE,),
        in_specs=[
            pl.BlockSpec((EBLK, 256), lambda i: (i, 0)),
            pl.BlockSpec((EBLK, 8), lambda i: (i, 0)),
            pl.BlockSpec((1, EBLK, 1), lambda i: (i, 0, 0)),
            pl.BlockSpec((80, 128), lambda i: (0, 0)),
            pl.BlockSpec((80, 128), lambda i: (0, 0)),
            pl.BlockSpec((8, 256), lambda i: (0, 0)),
            pl.BlockSpec((256, 512), lambda i: (0, 0)),
            pl.BlockSpec((1, 512), lambda i: (0, 0)),
        ],
        out_specs=pl.BlockSpec((B, 512), lambda i: (0, 0)),
        out_shape=jax.ShapeDtypeStruct((B, 512), jnp.float32),
    )(us, rel, dst3, batchp, isop, w2ar8, w2b, b2b)


def kernel(pos, edge_index, batch, W1a, b1a, W1b, b1b, W2a, b2a, W2b, b2b):
    src = edge_index[0]
    dst = edge_index[1]

    posp = jnp.zeros((N, 8), jnp.float32).at[:, :3].set(pos).reshape(N * 8)
    w1a8 = jnp.zeros((8, 64), jnp.float32).at[:3].set(W1a)
    w2ar8 = jnp.zeros((8, 256), jnp.float32).at[:3].set(W2a[128:131])

    rel = _sc_rel(posp, src, dst).reshape(E, 8)

    h = _mlp1(rel, w1a8, b1a.reshape(1, 64), W1b, b1b.reshape(1, 128))

    xp, iso = _sc_segment_max(h, dst)

    u2 = _tables(xp, W2a[:128], b2a.reshape(1, 256))

    us = _sc_gather_u2(u2, src)

    batchp = jnp.full((NP,), B, jnp.int32).at[:N].set(batch).reshape(80, 128)
    isop = iso.reshape(80, 128)
    dst3 = dst.reshape(NBE, EBLK, 1)

    return _mlp2_pool(us, rel, dst3, batchp, isop, w2ar8, W2b,
                      b2b.reshape(1, 512))


# speedup vs baseline: 1.5753x; 1.0245x over previous
import functools

import jax
import jax.numpy as jnp
from jax import lax
from jax.experimental import pallas as pl
from jax.experimental.pallas import tpu as pltpu
from jax.experimental.pallas import tpu_sc as plsc

N, E, B = 10000, 320000, 16
EBLK = 512
NBE = E // EBLK
NP = 10240
NBN = NP // 128
NW = 32
EPW = E // NW
GCH = 80
NCH_G = EPW // GCH
DCH = 8000
NCH_D = E // DCH
NODES_PW = NP // NW

NEG_INF = float("-inf")

_sc_mesh = lambda: plsc.VectorSubcoreMesh(
    core_axis_name="c", subcore_axis_name="s")


def _wid():
    return lax.axis_index("s") * 2 + lax.axis_index("c")


def _sc_rel_body(pos_hbm, src_hbm, dst_hbm, rel_hbm, postbl, sidx, didx, rbuf):
    base = _wid() * EPW
    pltpu.sync_copy(pos_hbm, postbl)

    def zgrp(gi, _):
        rbuf[pl.ds(gi * 16, 16)] = jnp.zeros((16,), jnp.float32)
        return 0

    lax.fori_loop(0, (GCH * 8) // 16, zgrp, 0)

    def chunk(ci, _):
        off = base + ci * GCH
        pltpu.sync_copy(src_hbm.at[pl.ds(off, GCH)], sidx)
        pltpu.sync_copy(dst_hbm.at[pl.ds(off, GCH)], didx)

        def grp(gi, _):
            ls = sidx[pl.ds(gi * 16, 16)] * 8
            ld = didx[pl.ds(gi * 16, 16)] * 8
            for k in range(3):
                sv = plsc.load_gather(postbl, [ls + k])
                dv = plsc.load_gather(postbl, [ld + k])
                fidx = jnp.arange(16, dtype=jnp.int32) * 8 + (gi * 128 + k)
                plsc.store_scatter(rbuf, [fidx], sv - dv)
            return 0

        lax.fori_loop(0, GCH // 16, grp, 0)
        pltpu.sync_copy(rbuf, rel_hbm.at[pl.ds(off * 8, GCH * 8)])
        return 0

    lax.fori_loop(0, NCH_G, chunk, 0)


def _sc_rel(posp, src, dst):
    f = functools.partial(
        pl.kernel,
        out_type=jax.ShapeDtypeStruct((E * 8,), jnp.float32),
        mesh=_sc_mesh(),
        compiler_params=pltpu.CompilerParams(needs_layout_passes=False),
        scratch_types=[
            pltpu.VMEM((N * 8,), jnp.float32),
            pltpu.VMEM((GCH,), jnp.int32),
            pltpu.VMEM((GCH,), jnp.int32),
            pltpu.VMEM((GCH * 8,), jnp.float32),
        ],
    )(_sc_rel_body)
    return f(posp, src, dst)


KMASK = (1 << 19) - 1


def _sc_segment_max_body(h_hbm, dst_hbm, x_hbm, iso_hbm,
                         dbuf0, dbuf1, cidx, gidx, hrows0, hrows1, acc,
                         ibuf, dsem0, dsem1, gsem0, gsem1):
    wid = _wid()
    lo = wid * NODES_PW
    hi = lo + NODES_PW

    def initrow(r, _):
        for k in range(8):
            acc[r, pl.ds(k * 16, 16)] = jnp.full((16,), NEG_INF, jnp.float32)
        return 0

    lax.fori_loop(0, NODES_PW, initrow, 0)

    def initcidx(gi, _):
        cidx[pl.ds(gi * 16, 16)] = jnp.zeros((16,), jnp.int32)
        return 0

    lax.fori_loop(0, (DCH + 128) // 16, initcidx, 0)

    dbufs = (dbuf0, dbuf1)
    dsems = (dsem0, dsem1)
    pltpu.async_copy(dst_hbm.at[pl.ds(0, DCH)], dbuf0, dsem0)
    pltpu.async_copy(dst_hbm.at[pl.ds(DCH, DCH)], dbuf1, dsem1)

    def pair(cj, _):
        for b in range(2):
            ci = cj * 2 + b
            eoff = ci * DCH
            dbufb = dbufs[b]
            pltpu.make_async_copy(
                dst_hbm.at[pl.ds(eoff, DCH)], dbufb, dsems[b]).wait()

            def grp(gi, cnt):
                d = dbufb[pl.ds(gi * 16, 16)]
                m = (d >= lo) & (d < hi)
                eid = jnp.arange(16, dtype=jnp.int32) + (eoff + gi * 16)
                key = ((d - lo) << 19) | eid
                plsc.store_compressed(cidx.at[pl.ds(cnt, 16)], key, mask=m)
                return cnt + plsc.all_reduce_population_count(m)[0]

            cnt = lax.fori_loop(0, DCH // 16, grp, 0)

            @pl.when(ci + 2 < NCH_D)
            def _prefetch():
                pltpu.async_copy(
                    dst_hbm.at[pl.ds(eoff + 2 * DCH, DCH)], dbufb, dsems[b])

            npieces = (cnt + 127) // 128

            def conv(gi, _):
                kv = cidx[pl.ds(gi * 16, 16)]
                gidx[pl.ds(gi * 16, 16)] = kv & KMASK
                return 0

            lax.fori_loop(0, npieces * 8, conv, 0)

            hbufs = (hrows0, hrows1)
            gsems = (gsem0, gsem1)

            @pl.when(npieces > 0)
            def _fire0():
                pltpu.async_copy(
                    h_hbm.at[gidx.at[pl.ds(0, 128)]], hrows0, gsem0)

            def _rmw_from(pi, hb):
                mj = jnp.minimum(cnt - pi * 128, 128)

                def rmw(j, _):
                    key = cidx[pl.ds(pi * 128 + j, 16)][0]
                    nrow = key >> 19
                    for k in range(8):
                        sl = pl.ds(k * 16, 16)
                        acc[nrow, sl] = jnp.maximum(acc[nrow, sl],
                                                    hb[j, sl])
                    return 0

                lax.fori_loop(0, mj, rmw, 0)

            def piece(pi, _):
                even = (pi & 1) == 0
                for par in range(2):
                    nb = 1 - par

                    @pl.when((pi + 1 < npieces) & (even == (par == 0)))
                    def _fire_next():
                        pltpu.async_copy(
                            h_hbm.at[gidx.at[pl.ds((pi + 1) * 128, 128)]],
                            hbufs[nb], gsems[nb])

                for par in range(2):
                    @pl.when(even == (par == 0))
                    def _drain_rmw():
                        pltpu.make_async_copy(
                            h_hbm.at[gidx.at[pl.ds(pi * 128, 128)]],
                            hbufs[par], gsems[par]).wait()
                        _rmw_from(pi, hbufs[par])

                return 0

            lax.fori_loop(0, npieces, piece, 0)
        return 0

    lax.fori_loop(0, NCH_D // 2, pair, 0)

    def fingrp(gi, _):
        rvec = jnp.arange(16, dtype=jnp.int32) + gi * 16
        vals = plsc.load_gather(acc, [rvec, jnp.zeros((16,), jnp.int32)])
        flags = jnp.where(vals == NEG_INF, 1, 0).astype(jnp.int32)
        ibuf[pl.ds(gi * 16, 16)] = flags
        return 0

    lax.fori_loop(0, NODES_PW // 16, fingrp, 0)

    def finrow(r, _):
        for k in range(8):
            sl = pl.ds(k * 16, 16)
            w = acc[r, sl]
            acc[r, sl] = jnp.where(w == NEG_INF, 0.0, w)
        return 0

    lax.fori_loop(0, NODES_PW, finrow, 0)
    pltpu.sync_copy(acc, x_hbm.at[pl.ds(lo, NODES_PW)])
    pltpu.sync_copy(ibuf, iso_hbm.at[pl.ds(lo, NODES_PW)])


def _sc_segment_max(h, dst):
    f = functools.partial(
        pl.kernel,
        out_type=[jax.ShapeDtypeStruct((NP, 128), jnp.float32),
                  jax.ShapeDtypeStruct((NP,), jnp.int32)],
        mesh=_sc_mesh(),
        compiler_params=pltpu.CompilerParams(needs_layout_passes=False),
        scratch_types=[
            pltpu.VMEM((DCH,), jnp.int32),
            pltpu.VMEM((DCH,), jnp.int32),
            pltpu.VMEM((DCH + 128,), jnp.int32),
            pltpu.VMEM((DCH + 128,), jnp.int32),
            pltpu.VMEM((128, 128), jnp.float32),
            pltpu.VMEM((128, 128), jnp.float32),
            pltpu.VMEM((NODES_PW, 128), jnp.float32),
            pltpu.VMEM((NODES_PW,), jnp.int32),
            pltpu.SemaphoreType.DMA,
            pltpu.SemaphoreType.DMA,
            pltpu.SemaphoreType.DMA,
            pltpu.SemaphoreType.DMA,
        ],
    )(_sc_segment_max_body)
    return f(h, dst)


def _sc_gather_u2_body(u2_hbm, src_hbm, us_hbm, sidx, ubuf, sem):
    base = _wid() * EPW

    def chunk(ci, _):
        off = base + ci * GCH
        pltpu.sync_copy(src_hbm.at[pl.ds(off, GCH)], sidx)
        pltpu.async_copy(u2_hbm.at[sidx], ubuf, sem).wait()
        pltpu.sync_copy(ubuf, us_hbm.at[pl.ds(off, GCH)])
        return 0

    lax.fori_loop(0, NCH_G, chunk, 0)


def _sc_gather_u2(u2, src):
    f = functools.partial(
        pl.kernel,
        out_type=jax.ShapeDtypeStruct((E, 256), jnp.float32),
        mesh=_sc_mesh(),
        compiler_params=pltpu.CompilerParams(needs_layout_passes=False),
        scratch_types=[
            pltpu.VMEM((GCH,), jnp.int32),
            pltpu.VMEM((GCH, 256), jnp.float32),
            pltpu.SemaphoreType.DMA,
        ],
    )(_sc_gather_u2_body)
    return f(u2, src)


def _mlp1_body(rel_ref, w1a_ref, b1a_ref, w1b_ref, b1b_ref, out_ref):
    relb = rel_ref[...]
    t = jnp.dot(relb, w1a_ref[...], preferred_element_type=jnp.float32)
    t = jnp.maximum(t + b1a_ref[...], 0.0)
    h = jnp.dot(t, w1b_ref[...], preferred_element_type=jnp.float32)
    out_ref[...] = h + b1b_ref[...]


def _mlp1(rel, w1a8, b1a, w1b, b1b):
    return pl.pallas_call(
        _mlp1_body,
        grid=(NBE,),
        in_specs=[
            pl.BlockSpec((EBLK, 8), lambda i: (i, 0)),
            pl.BlockSpec((8, 64), lambda i: (0, 0)),
            pl.BlockSpec((1, 64), lambda i: (0, 0)),
            pl.BlockSpec((64, 128), lambda i: (0, 0)),
            pl.BlockSpec((1, 128), lambda i: (0, 0)),
        ],
        out_specs=pl.BlockSpec((EBLK, 128), lambda i: (i, 0)),
        out_shape=jax.ShapeDtypeStruct((E, 128), jnp.float32),
    )(rel, w1a8, b1a, w1b, b1b)


def _tables_body(x_ref, w2ax_ref, b2a_ref, u_ref):
    ub = jnp.dot(x_ref[...], w2ax_ref[...], preferred_element_type=jnp.float32)
    u_ref[...] = ub + b2a_ref[...]


def _tables(xp, w2ax, b2a):
    return pl.pallas_call(
        _tables_body,
        grid=(NBN,),
        in_specs=[
            pl.BlockSpec((128, 128), lambda i: (i, 0)),
            pl.BlockSpec((128, 256), lambda i: (0, 0)),
            pl.BlockSpec((1, 256), lambda i: (0, 0)),
        ],
        out_specs=pl.BlockSpec((128, 256), lambda i: (i, 0)),
        out_shape=jax.ShapeDtypeStruct((NP, 256), jnp.float32),
    )(xp, w2ax, b2a)


def _mlp2_body(us_ref, rel_ref, dst_ref, batch_ref, iso_ref,
               w2ar_ref, w2b_ref, b2b_ref, g_ref):
    i = pl.program_id(0)

    @pl.when(i == 0)
    def _init():
        g_ref[...] = jnp.full((B, 512), NEG_INF, jnp.float32)

    relb = rel_ref[...]
    a = us_ref[...] + jnp.dot(relb, w2ar_ref[...],
                              preferred_element_type=jnp.float32)
    v = jnp.maximum(a, 0.0)
    h2 = jnp.dot(v, w2b_ref[...], preferred_element_type=jnp.float32)
    h2 = h2 + b2b_ref[...]
    dst = dst_ref[0]
    batch = batch_ref[...]

    lbl = jnp.zeros_like(dst)
    for b in range(1, B):
        start_b = jnp.sum((batch < b).astype(jnp.int32))
        lbl = lbl + (dst >= start_b).astype(jnp.int32)

    h2b = h2.astype(jnp.bfloat16)
    ninf_b = jnp.asarray(NEG_INF, jnp.bfloat16)
    acc = g_ref[...]
    rows = []
    for b in range(B):
        mb = jnp.max(jnp.where(lbl == b, h2b, ninf_b), axis=0, keepdims=True)
        rows.append(mb.astype(jnp.float32))
    g_ref[...] = jnp.maximum(acc, jnp.concatenate(rows, axis=0))

    @pl.when(i == NBE - 1)
    def _fin():
        iso = iso_ref[...]
        g = g_ref[...]
        floors = []
        for b in range(B):
            has_iso = jnp.sum(iso * (batch == b).astype(jnp.int32)) > 0
            floors.append(jnp.where(has_iso, 0.0, NEG_INF).reshape(1, 1))
        g = jnp.maximum(g, jnp.concatenate(floors, axis=0))
        g_ref[...] = jnp.where(jnp.isfinite(g), g, 0.0)


def _mlp2_pool(us, rel, dst3, batchp, isop, w2ar8, w2b, b2b):
    return pl.pallas_call(
        _mlp2_body,
        grid=(NBE,),
        in_specs=[
            pl.BlockSpec((EBLK, 256), lambda i: (i, 0)),
            pl.BlockSpec((EBLK, 8), lambda i: (i, 0)),
            pl.BlockSpec((1, EBLK, 1), lambda i: (i, 0, 0)),
            pl.BlockSpec((80, 128), lambda i: (0, 0)),
            pl.BlockSpec((80, 128), lambda i: (0, 0)),
            pl.BlockSpec((8, 256), lambda i: (0, 0)),
            pl.BlockSpec((256, 512), lambda i: (0, 0)),
            pl.BlockSpec((1, 512), lambda i: (0, 0)),
        ],
        out_specs=pl.BlockSpec((B, 512), lambda i: (0, 0)),
        out_shape=jax.ShapeDtypeStruct((B, 512), jnp.float32),
    )(us, rel, dst3, batchp, isop, w2ar8, w2b, b2b)


def kernel(pos, edge_index, batch, W1a, b1a, W1b, b1b, W2a, b2a, W2b, b2b):
    src = edge_index[0]
    dst = edge_index[1]

    posp = jnp.zeros((N, 8), jnp.float32).at[:, :3].set(pos).reshape(N * 8)
    w1a8 = jnp.zeros((8, 64), jnp.float32).at[:3].set(W1a)
    w2ar8 = jnp.zeros((8, 256), jnp.float32).at[:3].set(W2a[128:131])

    rel = _sc_rel(posp, src, dst).reshape(E, 8)

    h = _mlp1(rel, w1a8, b1a.reshape(1, 64), W1b, b1b.reshape(1, 128))

    xp, iso = _sc_segment_max(h, dst)

    u2 = _tables(xp, W2a[:128], b2a.reshape(1, 256))

    us = _sc_gather_u2(u2, src)

    batchp = jnp.full((NP,), B, jnp.int32).at[:N].set(batch).reshape(80, 128)
    isop = iso.reshape(80, 128)
    dst3 = dst.reshape(NBE, EBLK, 1)

    return _mlp2_pool(us, rel, dst3, batchp, isop, w2ar8, W2b,
                      b2b.reshape(1, 512))
